# Initial kernel scaffold; baseline (speedup 1.0000x reference)
import jax
import jax.numpy as jnp
from jax.experimental import pallas as pl


def kernel(x, edge_index, batch, W1l, b1, W1r, g1, be1, W2l, b2, W2r, g2, be2, W3l, b3, W3r, g3, be3, f1W, f1b, f2W, f2b, f3W, f3b):
    raise NotImplementedError("write your pallas kernel here")


# speedup vs baseline: 6.6225x; 6.6225x over previous
import functools

import jax
import jax.numpy as jnp
from jax import lax
from jax.experimental import pallas as pl
from jax.experimental.pallas import tpu as pltpu
from jax.experimental.pallas import tpu_sc as plsc

NC = 2
NS = 16
NW = NC * NS
K = 128


@functools.partial(jax.jit, static_argnames=("n_pad", "w", "with_cnt"))
def _sc_scatter(p, src, dst, zrow, zcnt, ones, *, n_pad, w, with_cnt):
    e = src.shape[0]
    tch = e // K
    base = tch // NW
    extra = tch % NW
    rpt = n_pad // NS

    mesh = plsc.VectorSubcoreMesh(core_axis_name="c", subcore_axis_name="s")
    out_type = [jax.ShapeDtypeStruct((NC * n_pad, w), jnp.float32)]
    scratch = [
        pltpu.VMEM((K,), jnp.int32),
        pltpu.VMEM((K,), jnp.int32),
        pltpu.VMEM((K, w), jnp.float32),
        pltpu.VMEM_SHARED((n_pad, w), jnp.float32),
        pltpu.SemaphoreType.DMA,
    ]
    if with_cnt:
        out_type.append(jax.ShapeDtypeStruct((NC * n_pad, 16), jnp.float32))
        scratch += [
            pltpu.VMEM((K, 16), jnp.float32),
            pltpu.VMEM_SHARED((n_pad, 16), jnp.float32),
        ]

    @functools.partial(
        pl.kernel, mesh=mesh, out_type=out_type, scratch_types=scratch,
        compiler_params=pltpu.CompilerParams(use_tc_tiling_on_sc=False))
    def body(p_hbm, src_hbm, dst_hbm, zrow_hbm, zcnt_hbm, ones_hbm,
             *refs):
        if with_cnt:
            (s_out, c_out, src_v, dst_v, rows_v, s_sh, gsem,
             ones_v, c_sh) = refs
        else:
            (s_out, src_v, dst_v, rows_v, s_sh, gsem) = refs
        c = lax.axis_index("c")
        s = lax.axis_index("s")
        t = c * NS + s

        pltpu.sync_copy(zrow_hbm, s_sh.at[pl.ds(s * rpt, rpt)])
        if with_cnt:
            pltpu.sync_copy(zcnt_hbm, c_sh.at[pl.ds(s * rpt, rpt)])
            pltpu.sync_copy(ones_hbm, ones_v)
        plsc.subcore_barrier()

        nch = base + jnp.where(t < extra, 1, 0)
        start = t * base + jnp.minimum(t, extra)

        def step(i, carry):
            off = (start + i) * K
            pltpu.sync_copy(src_hbm.at[pl.ds(off, K)], src_v)
            pltpu.sync_copy(dst_hbm.at[pl.ds(off, K)], dst_v)
            pltpu.async_copy(p_hbm.at[src_v], rows_v, gsem).wait()
            pltpu.sync_copy(rows_v, s_sh.at[dst_v], add=True)
            if with_cnt:
                pltpu.sync_copy(ones_v, c_sh.at[dst_v], add=True)
            return carry

        lax.fori_loop(0, nch, step, 0)
        plsc.subcore_barrier()

        row0 = c * n_pad + s * rpt
        pltpu.sync_copy(s_sh.at[pl.ds(s * rpt, rpt)],
                        s_out.at[pl.ds(row0, rpt)])
        if with_cnt:
            pltpu.sync_copy(c_sh.at[pl.ds(s * rpt, rpt)],
                            c_out.at[pl.ds(row0, rpt)])

    if with_cnt:
        return body(p, src, dst, zrow, zcnt, ones)
    return body(p, src, dst, zrow, zcnt, ones)


def _tc_matmul(x, w):
    def body(x_ref, w_ref, o_ref):
        o_ref[...] = jnp.dot(x_ref[...], w_ref[...],
                             preferred_element_type=jnp.float32)
    return pl.pallas_call(
        body,
        out_shape=jax.ShapeDtypeStruct((x.shape[0], w.shape[1]), jnp.float32),
    )(x, w)


def _tc_mid(s2, c2, pq, b, g, be, wnext, *, n, n_pad, w):
    def body(s_ref, c_ref, pq_ref, b_ref, g_ref, be_ref, w_ref, o_ref):
        ssum = s_ref[0:n, :] + s_ref[n_pad:n_pad + n, :]
        cnt = c_ref[0:n, 0:1] + c_ref[n_pad:n_pad + n, 0:1]
        a = ssum / jnp.maximum(cnt, 1.0) + b_ref[...] + pq_ref[:, w:]
        mu = jnp.mean(a, axis=0, keepdims=True)
        var = jnp.mean((a - mu) * (a - mu), axis=0, keepdims=True)
        h = (a - mu) * lax.rsqrt(var + 1e-5) * g_ref[...] + be_ref[...]
        h = jnp.maximum(h, 0.0)
        o_ref[...] = jnp.dot(h, w_ref[...], preferred_element_type=jnp.float32)

    return pl.pallas_call(
        body,
        out_shape=jax.ShapeDtypeStruct((n, wnext.shape[1]), jnp.float32),
    )(s2, c2, pq, b.reshape(1, -1), g.reshape(1, -1), be.reshape(1, -1), wnext)


def _tc_final(s2, c2, pq, b, g, be, batch2, f1w, f1b, f2w, f2b, f3w, f3b,
              *, n, n_pad, w, g_groups):
    def body(s_ref, c_ref, pq_ref, b_ref, g_ref, be_ref, batch_ref,
             f1w_ref, f1b_ref, f2w_ref, f2b_ref, f3w_ref, f3b_ref, o_ref):
        ssum = s_ref[0:n, :] + s_ref[n_pad:n_pad + n, :]
        cnt = c_ref[0:n, 0:1] + c_ref[n_pad:n_pad + n, 0:1]
        a = ssum / jnp.maximum(cnt, 1.0) + b_ref[...] + pq_ref[:, w:]
        mu = jnp.mean(a, axis=0, keepdims=True)
        var = jnp.mean((a - mu) * (a - mu), axis=0, keepdims=True)
        h = (a - mu) * lax.rsqrt(var + 1e-5) * g_ref[...] + be_ref[...]
        h = jnp.maximum(h, 0.0)

        gid = lax.broadcasted_iota(jnp.int32, (g_groups, n), 0)
        onehot = (gid == batch_ref[...]).astype(jnp.float32)
        gsum = jnp.dot(onehot, h, preferred_element_type=jnp.float32)
        gcnt = jnp.sum(onehot, axis=1, keepdims=True)
        hp = gsum / jnp.maximum(gcnt, 1.0)

        hp = jnp.maximum(jnp.dot(hp, f1w_ref[...],
                                 preferred_element_type=jnp.float32)
                         + f1b_ref[...], 0.0)
        hp = jnp.maximum(jnp.dot(hp, f2w_ref[...],
                                 preferred_element_type=jnp.float32)
                         + f2b_ref[...], 0.0)
        o_ref[...] = jnp.dot(hp, f3w_ref[...],
                             preferred_element_type=jnp.float32) + f3b_ref[...]

    return pl.pallas_call(
        body,
        out_shape=jax.ShapeDtypeStruct((g_groups, f3w.shape[1]), jnp.float32),
    )(s2, c2, pq, b.reshape(1, -1), g.reshape(1, -1), be.reshape(1, -1),
      batch2, f1w, f1b.reshape(1, -1), f2w, f2b.reshape(1, -1), f3w,
      f3b.reshape(1, -1))


def kernel(x, edge_index, batch, W1l, b1, W1r, g1, be1, W2l, b2, W2r, g2, be2,
           W3l, b3, W3r, g3, be3, f1W, f1b, f2W, f2b, f3W, f3b):
    n, d = x.shape
    n_pad = ((n + NS * 8 - 1) // (NS * 8)) * (NS * 8)
    rpt = n_pad // NS
    src = edge_index[0]
    dst = edge_index[1]
    g_groups = 64

    h1 = W1l.shape[1]
    h2 = W2l.shape[1]
    h3 = W3l.shape[1]

    zrow1 = jnp.zeros((rpt, h1), jnp.float32)
    zrow2 = jnp.zeros((rpt, h2), jnp.float32)
    zrow3 = jnp.zeros((rpt, h3), jnp.float32)
    zcnt = jnp.zeros((rpt, 16), jnp.float32)
    ones = jnp.ones((K, 16), jnp.float32)
    batch2 = batch.reshape(1, n)

    pq1 = _tc_matmul(x, jnp.concatenate([W1l, W1r], axis=1))
    s1, c1 = _sc_scatter(pq1[:, :h1], src, dst, zrow1, zcnt, ones,
                         n_pad=n_pad, w=h1, with_cnt=True)
    pq2 = _tc_mid(s1, c1, pq1, b1, g1, be1,
                  jnp.concatenate([W2l, W2r], axis=1),
                  n=n, n_pad=n_pad, w=h1)
    (s2,) = _sc_scatter(pq2[:, :h2], src, dst, zrow2, zcnt, ones,
                        n_pad=n_pad, w=h2, with_cnt=False)
    pq3 = _tc_mid(s2, c1, pq2, b2, g2, be2,
                  jnp.concatenate([W3l, W3r], axis=1),
                  n=n, n_pad=n_pad, w=h2)
    (s3,) = _sc_scatter(pq3[:, :h3], src, dst, zrow3, zcnt, ones,
                        n_pad=n_pad, w=h3, with_cnt=False)
    return _tc_final(s3, c1, pq3, b3, g3, be3, batch2,
                     f1W, f1b, f2W, f2b, f3W, f3b,
                     n=n, n_pad=n_pad, w=h3, g_groups=g_groups)


# speedup vs baseline: 12.3227x; 1.8607x over previous
import functools

import jax
import jax.numpy as jnp
from jax import lax
from jax.experimental import pallas as pl
from jax.experimental.pallas import tpu as pltpu
from jax.experimental.pallas import tpu_sc as plsc

NC = 2
NS = 16
NW = NC * NS
K = 128


@functools.partial(jax.jit, static_argnames=("n_pad", "w", "with_cnt"))
def _sc_scatter(p, src2, dst2, zrow, zcnt, ones, *, n_pad, w, with_cnt):
    tch = src2.shape[0]
    gk = 1 if w > 64 else (5 if w > 32 else 10)
    ngroups = tch // gk
    base = ngroups // NW
    extra = ngroups % NW
    rpt = n_pad // NS

    mesh = plsc.VectorSubcoreMesh(core_axis_name="c", subcore_axis_name="s")
    out_type = [jax.ShapeDtypeStruct((NC * n_pad, w), jnp.float32)]
    scratch = [
        pltpu.VMEM((2, gk, K), jnp.int32),
        pltpu.VMEM((2, gk, K), jnp.int32),
        pltpu.VMEM((2, gk, K, w), jnp.float32),
        pltpu.VMEM_SHARED((n_pad, w), jnp.float32),
        pltpu.SemaphoreType.DMA,
    ]
    if with_cnt:
        out_type.append(jax.ShapeDtypeStruct((NC * n_pad, 16), jnp.float32))
        scratch += [
            pltpu.VMEM((K, 16), jnp.float32),
            pltpu.VMEM_SHARED((n_pad, 16), jnp.float32),
        ]

    @functools.partial(
        pl.kernel, mesh=mesh, out_type=out_type, scratch_types=scratch,
        compiler_params=pltpu.CompilerParams(use_tc_tiling_on_sc=False))
    def body(p_hbm, src_hbm, dst_hbm, zrow_hbm, zcnt_hbm, ones_hbm,
             *refs):
        if with_cnt:
            (s_out, c_out, src_v, dst_v, rows_v, s_sh, gsem,
             ones_v, c_sh) = refs
        else:
            (s_out, src_v, dst_v, rows_v, s_sh, gsem) = refs
        c = lax.axis_index("c")
        s = lax.axis_index("s")
        t = c * NS + s

        pltpu.sync_copy(zrow_hbm, s_sh.at[pl.ds(s * rpt, rpt)])
        if with_cnt:
            pltpu.sync_copy(zcnt_hbm, c_sh.at[pl.ds(s * rpt, rpt)])
            pltpu.sync_copy(ones_hbm, ones_v)
        plsc.subcore_barrier()

        ng = base + jnp.where(t < extra, 1, 0)
        g0 = t * base + jnp.minimum(t, extra)

        def load_and_fire(b, g):
            chunk0 = (g0 + g) * gk
            pltpu.sync_copy(src_hbm.at[pl.ds(chunk0, gk)], src_v.at[b])
            pltpu.sync_copy(dst_hbm.at[pl.ds(chunk0, gk)], dst_v.at[b])
            for j in range(gk):
                pltpu.async_copy(p_hbm.at[src_v.at[b, j]], rows_v.at[b, j],
                                 gsem)

        @pl.when(ng > 0)
        def _():
            load_and_fire(0, 0)

        def step(g, carry):
            b = lax.rem(g, 2)

            @pl.when(g + 1 < ng)
            def _():
                load_and_fire(1 - b, g + 1)

            for j in range(gk):
                pltpu.make_async_copy(p_hbm.at[src_v.at[b, j]],
                                      rows_v.at[b, j], gsem).wait()
            for j in range(gk):
                pltpu.sync_copy(rows_v.at[b, j], s_sh.at[dst_v.at[b, j]],
                                add=True)
                if with_cnt:
                    pltpu.sync_copy(ones_v, c_sh.at[dst_v.at[b, j]],
                                    add=True)
            return carry

        lax.fori_loop(0, ng, step, 0)
        plsc.subcore_barrier()

        row0 = c * n_pad + s * rpt
        pltpu.sync_copy(s_sh.at[pl.ds(s * rpt, rpt)],
                        s_out.at[pl.ds(row0, rpt)])
        if with_cnt:
            pltpu.sync_copy(c_sh.at[pl.ds(s * rpt, rpt)],
                            c_out.at[pl.ds(row0, rpt)])

    return body(p, src2, dst2, zrow, zcnt, ones)


def _tc_matmul(x, w):
    def body(x_ref, w_ref, o_ref):
        o_ref[...] = jnp.dot(x_ref[...], w_ref[...],
                             preferred_element_type=jnp.float32)
    return pl.pallas_call(
        body,
        out_shape=jax.ShapeDtypeStruct((x.shape[0], w.shape[1]), jnp.float32),
    )(x, w)


def _tc_mid(s2, c2, pq, b, g, be, wnext, *, n, n_pad, w):
    def body(s_ref, c_ref, pq_ref, b_ref, g_ref, be_ref, w_ref, o_ref):
        ssum = s_ref[0:n, :] + s_ref[n_pad:n_pad + n, :]
        cnt = c_ref[0:n, 0:1] + c_ref[n_pad:n_pad + n, 0:1]
        a = ssum / jnp.maximum(cnt, 1.0) + b_ref[...] + pq_ref[:, w:]
        mu = jnp.mean(a, axis=0, keepdims=True)
        var = jnp.mean((a - mu) * (a - mu), axis=0, keepdims=True)
        h = (a - mu) * lax.rsqrt(var + 1e-5) * g_ref[...] + be_ref[...]
        h = jnp.maximum(h, 0.0)
        o_ref[...] = jnp.dot(h, w_ref[...], preferred_element_type=jnp.float32)

    return pl.pallas_call(
        body,
        out_shape=jax.ShapeDtypeStruct((n, wnext.shape[1]), jnp.float32),
    )(s2, c2, pq, b.reshape(1, -1), g.reshape(1, -1), be.reshape(1, -1), wnext)


def _tc_final(s2, c2, pq, b, g, be, batch2, f1w, f1b, f2w, f2b, f3w, f3b,
              *, n, n_pad, w, g_groups):
    def body(s_ref, c_ref, pq_ref, b_ref, g_ref, be_ref, batch_ref,
             f1w_ref, f1b_ref, f2w_ref, f2b_ref, f3w_ref, f3b_ref, o_ref):
        ssum = s_ref[0:n, :] + s_ref[n_pad:n_pad + n, :]
        cnt = c_ref[0:n, 0:1] + c_ref[n_pad:n_pad + n, 0:1]
        a = ssum / jnp.maximum(cnt, 1.0) + b_ref[...] + pq_ref[:, w:]
        mu = jnp.mean(a, axis=0, keepdims=True)
        var = jnp.mean((a - mu) * (a - mu), axis=0, keepdims=True)
        h = (a - mu) * lax.rsqrt(var + 1e-5) * g_ref[...] + be_ref[...]
        h = jnp.maximum(h, 0.0)

        gid = lax.broadcasted_iota(jnp.int32, (g_groups, n), 0)
        onehot = (gid == batch_ref[...]).astype(jnp.float32)
        gsum = jnp.dot(onehot, h, preferred_element_type=jnp.float32)
        gcnt = jnp.sum(onehot, axis=1, keepdims=True)
        hp = gsum / jnp.maximum(gcnt, 1.0)

        hp = jnp.maximum(jnp.dot(hp, f1w_ref[...],
                                 preferred_element_type=jnp.float32)
                         + f1b_ref[...], 0.0)
        hp = jnp.maximum(jnp.dot(hp, f2w_ref[...],
                                 preferred_element_type=jnp.float32)
                         + f2b_ref[...], 0.0)
        o_ref[...] = jnp.dot(hp, f3w_ref[...],
                             preferred_element_type=jnp.float32) + f3b_ref[...]

    return pl.pallas_call(
        body,
        out_shape=jax.ShapeDtypeStruct((g_groups, f3w.shape[1]), jnp.float32),
    )(s2, c2, pq, b.reshape(1, -1), g.reshape(1, -1), be.reshape(1, -1),
      batch2, f1w, f1b.reshape(1, -1), f2w, f2b.reshape(1, -1), f3w,
      f3b.reshape(1, -1))


def kernel(x, edge_index, batch, W1l, b1, W1r, g1, be1, W2l, b2, W2r, g2, be2,
           W3l, b3, W3r, g3, be3, f1W, f1b, f2W, f2b, f3W, f3b):
    n, d = x.shape
    n_pad = ((n + NS * 8 - 1) // (NS * 8)) * (NS * 8)
    rpt = n_pad // NS
    e = edge_index.shape[1]
    src = edge_index[0].reshape(e // K, K)
    dst = edge_index[1].reshape(e // K, K)
    g_groups = 64

    h1 = W1l.shape[1]
    h2 = W2l.shape[1]
    h3 = W3l.shape[1]

    zrow1 = jnp.zeros((rpt, h1), jnp.float32)
    zrow2 = jnp.zeros((rpt, h2), jnp.float32)
    zrow3 = jnp.zeros((rpt, h3), jnp.float32)
    zcnt = jnp.zeros((rpt, 16), jnp.float32)
    ones = jnp.ones((K, 16), jnp.float32)
    batch2 = batch.reshape(1, n)

    pq1 = _tc_matmul(x, jnp.concatenate([W1l, W1r], axis=1))
    s1, c1 = _sc_scatter(pq1[:, :h1], src, dst, zrow1, zcnt, ones,
                         n_pad=n_pad, w=h1, with_cnt=True)
    pq2 = _tc_mid(s1, c1, pq1, b1, g1, be1,
                  jnp.concatenate([W2l, W2r], axis=1),
                  n=n, n_pad=n_pad, w=h1)
    (s2,) = _sc_scatter(pq2[:, :h2], src, dst, zrow2, zcnt, ones,
                        n_pad=n_pad, w=h2, with_cnt=False)
    pq3 = _tc_mid(s2, c1, pq2, b2, g2, be2,
                  jnp.concatenate([W3l, W3r], axis=1),
                  n=n, n_pad=n_pad, w=h2)
    (s3,) = _sc_scatter(pq3[:, :h3], src, dst, zrow3, zcnt, ones,
                        n_pad=n_pad, w=h3, with_cnt=False)
    return _tc_final(s3, c1, pq3, b3, g3, be3, batch2,
                     f1W, f1b, f2W, f2b, f3W, f3b,
                     n=n, n_pad=n_pad, w=h3, g_groups=g_groups)
